# baseline (speedup 1.0000x reference)
import functools

import jax
import jax.numpy as jnp
from jax import lax
from jax.experimental import pallas as pl
from jax.experimental.pallas import tpu as pltpu
from jax.experimental.pallas import tpu_sc as plsc

N = 50000
E = 800000
NP = 50176
BLK = 512
NBLK = NP // BLK

AR = 512
ATRASH = 65024

SUP = 2048
CKS = 128
NCK = SUP // CKS
NBUF = 2
GRP = NBUF * CKS
CCAP = SUP + GRP
NSUP = 26
EPT_B = NSUP * SUP
EPB = 16 * EPT_B
EPT_A = EPB // 32
CHA = 2048
NCH_A = EPT_A // CHA

NPASS = 3
RANGE = 8448
OUTR = 2 * NPASS * RANGE
TRASH = 256
SR = RANGE + TRASH
ZPT = SR // 16

_BN_SCALE = 1.0 / (1.0 + 1e-5) ** 0.5


def _segsum_scalar(srcp, dstp, xp):
  mesh = plsc.VectorSubcoreMesh(core_axis_name="c", subcore_axis_name="s")

  @functools.partial(
      pl.kernel,
      out_type=jax.ShapeDtypeStruct((2, AR, 128), jnp.float32),
      mesh=mesh,
      compiler_params=pltpu.CompilerParams(needs_layout_passes=False),
      scratch_types=[
          pltpu.VMEM((NP // 128, 128), jnp.float32),
          pltpu.VMEM((AR, 128), jnp.float32),
          pltpu.VMEM((CHA,), jnp.int32),
          pltpu.VMEM((CHA,), jnp.int32),
          pltpu.VMEM((4, 128), jnp.int32),
          pltpu.VMEM_SHARED((AR, 128), jnp.float32),
      ],
  )
  def body(src_hbm, dst_hbm, x_hbm, out_hbm, x_v, acc_v, sv, dv, ramp, acc_sh):
    cid = lax.axis_index("c")
    sid = lax.axis_index("s")
    wid = sid * 2 + cid
    pltpu.sync_copy(x_hbm, x_v)

    zeros16 = jnp.zeros((16,), jnp.float32)

    def zero_row(r, carry):
      for q in range(8):
        acc_v[r, pl.ds(q * 16, 16)] = zeros16
      return carry

    lax.fori_loop(0, AR, zero_row, 0)

    for g in range(4):
      for j in range(8):
        ramp[g, pl.ds(j * 16, 16)] = lax.iota(jnp.int32, 16) + (g * 128 + j * 16)

    ebase = wid * EPT_A

    def chunk(ch, carry):
      off = pl.multiple_of(ebase + ch * CHA, 8)
      pltpu.sync_copy(src_hbm.at[pl.ds(off, CHA)], sv)
      pltpu.sync_copy(dst_hbm.at[pl.ds(off, CHA)], dv)

      def inner(j, c2):
        s = sv[pl.ds(j * 16, 16)]
        d = dv[pl.ds(j * 16, 16)]
        d = jnp.where(d >= NP, ATRASH + jnp.bitwise_and(d, 127), d)
        vals = plsc.load_gather(
            x_v, [lax.shift_right_logical(s, 7), jnp.bitwise_and(s, 127)])
        plsc.addupdate_scatter(
            acc_v,
            [lax.shift_right_logical(d, 7), jnp.bitwise_and(d, 127)],
            vals,
        )
        return c2

      lax.fori_loop(0, CHA // 16, inner, 0)
      return carry

    lax.fori_loop(0, NCH_A, chunk, 0)

    @pl.when(sid == 0)
    def _():
      pltpu.sync_copy(acc_v, acc_sh)

    plsc.subcore_barrier()

    @pl.when(sid != 0)
    def _():
      for g in range(4):
        pltpu.sync_copy(
            acc_v.at[pl.ds(g * 128, 128)], acc_sh.at[ramp.at[g]], add=True
        )

    plsc.subcore_barrier()
    rows = AR // 16
    roff = pl.multiple_of(sid * rows, 8)
    pltpu.sync_copy(
        acc_sh.at[pl.ds(roff, rows)],
        out_hbm.at[cid, pl.ds(roff, rows)],
    )

  return body(srcp, dstp, xp.reshape(NP // 128, 128))


def _segsum_vec(srcp, dst2d, zeros_hbm, c1):
  mesh = plsc.VectorSubcoreMesh(core_axis_name="c", subcore_axis_name="s")

  @functools.partial(
      pl.kernel,
      out_type=jax.ShapeDtypeStruct((OUTR, 128), jnp.float32),
      mesh=mesh,
      compiler_params=pltpu.CompilerParams(needs_layout_passes=False),
      scratch_types=[
          pltpu.VMEM((SUP,), jnp.int32),
          pltpu.VMEM((NCK, CKS), jnp.int32),
          pltpu.VMEM((CCAP,), jnp.int32),
          pltpu.VMEM((CCAP,), jnp.int32),
          pltpu.VMEM((CKS, 128), jnp.float32),
          pltpu.VMEM((CKS, 128), jnp.float32),
          pltpu.VMEM((NBUF, CKS), jnp.int32),
          pltpu.VMEM_SHARED((SR, 128), jnp.float32),
          [pltpu.SemaphoreType.DMA] * NBUF,
      ],
  )
  def body(src_hbm, dst_hbm, z_hbm, c1_hbm, out_hbm, sv, dv, cs, cd,
           rows0, rows1, cd2, acc_sh, gsems):
    rows = [rows0, rows1]
    cid = lax.axis_index("c")
    sid = lax.axis_index("s")
    tb = sid * EPT_B

    def zcs(i, c):
      cs[pl.ds(i * 16, 16)] = jnp.zeros((16,), jnp.int32)
      return c
    lax.fori_loop(0, CCAP // 16, zcs, 0)

    for p in range(NPASS):
      base = (p * 2 + cid) * RANGE
      zoff = pl.multiple_of(sid * ZPT, 8)
      pltpu.sync_copy(z_hbm, acc_sh.at[pl.ds(zoff, ZPT)])
      plsc.subcore_barrier()

      def sup_body(sc, carry):
        off = pl.multiple_of(tb + sc * SUP, 8)
        pltpu.sync_copy(src_hbm.at[pl.ds(off, SUP)], sv)
        pltpu.sync_copy(dst_hbm.at[pl.ds(pl.multiple_of(off // CKS, 8), NCK)],
                        dv)

        def compact(g, cnt):
          r = g // (CKS // 16)
          q = g % (CKS // 16)
          d = dv[r, pl.ds(q * 16, 16)]
          s = sv[pl.ds(g * 16, 16)]
          l = d - base
          m = (l >= 0) & (l < RANGE)
          plsc.store_compressed(cs.at[pl.ds(cnt, 16)], s, mask=m)
          plsc.store_compressed(cd.at[pl.ds(cnt, 16)], l, mask=m)
          return cnt + jnp.max(plsc.all_reduce_population_count(m))

        cnt = lax.fori_loop(0, SUP // 16, compact, 0)

        tr16 = RANGE + lax.iota(jnp.int32, 16)
        for t in range(GRP // 16):
          cs[pl.ds(cnt + t * 16, 16)] = jnp.zeros((16,), jnp.int32)
          cd[pl.ds(cnt + t * 16, 16)] = tr16 + 16 * t

        ngroups = (cnt + GRP - 1) // GRP

        def gidx(grp, b):
          return pl.multiple_of(grp * GRP + b * CKS, 8)

        def grp_body(k4, c2):
          gs = [
              pltpu.async_copy(
                  c1_hbm.at[cs.at[pl.ds(gidx(k4, b), CKS)]], rows[b],
                  gsems[b])
              for b in range(NBUF)
          ]
          for b in range(NBUF):
            gs[b].wait()
            for j in range(CKS // 16):
              cd2[b, pl.ds(j * 16, 16)] = cd[pl.ds(gidx(k4, b) + j * 16, 16)]
            pltpu.sync_copy(rows[b], acc_sh.at[cd2.at[b]], add=True)
          return c2

        lax.fori_loop(0, ngroups, grp_body, 0)
        return carry

      lax.fori_loop(0, NSUP, sup_body, 0)
      plsc.subcore_barrier()
      doff = pl.multiple_of(sid * (RANGE // 16), 8)
      pltpu.sync_copy(
          acc_sh.at[pl.ds(doff, RANGE // 16)],
          out_hbm.at[pl.ds(pl.multiple_of(base + doff, 8), RANGE // 16)],
      )
      plsc.subcore_barrier()

  return body(srcp, dst2d, zeros_hbm, c1)


def _fuse_weights(W1b, W2a, b1b):
  def body(w1b_ref, w2a_ref, b1b_ref, wf_ref, bf_ref):
    wf_ref[...] = jnp.dot(w1b_ref[...], w2a_ref[...],
                          preferred_element_type=jnp.float32)
    bf_ref[...] = jnp.dot(b1b_ref[...], w2a_ref[...],
                          preferred_element_type=jnp.float32)

  return pl.pallas_call(
      body,
      out_shape=[
          jax.ShapeDtypeStruct((512, 128), jnp.float32),
          jax.ShapeDtypeStruct((1, 128), jnp.float32),
      ],
  )(W1b, W2a, b1b.reshape(1, 256))


def _tc1(xp2d, parts, W1a, b1a, g1, be1, Wf, bf):
  def body(x_ref, pp_ref, w1a_ref, b1a_ref, g1_ref, be1_ref, wf_ref, bf_ref,
           out_ref):
    pp = pp_ref[...]
    z = x_ref[...] + pp[0][:, None] + pp[1][:, None]
    t = z * w1a_ref[...] + b1a_ref[...]
    r = jnp.maximum(t * (g1_ref[...] * _BN_SCALE) + be1_ref[...], 0.0)
    out_ref[...] = (
        jnp.dot(r, wf_ref[...], preferred_element_type=jnp.float32)
        + bf_ref[...]
    )

  full = lambda shape: pl.BlockSpec(shape, lambda i: (0, 0))
  return pl.pallas_call(
      body,
      grid=(NBLK,),
      in_specs=[
          pl.BlockSpec((BLK, 1), lambda i: (i, 0)),
          pl.BlockSpec((2, BLK), lambda i: (0, i)),
          full((1, 512)),
          full((1, 512)),
          full((1, 512)),
          full((1, 512)),
          full((512, 128)),
          full((1, 128)),
      ],
      out_specs=pl.BlockSpec((BLK, 128), lambda i: (i, 0)),
      out_shape=jax.ShapeDtypeStruct((NP, 128), jnp.float32),
  )(xp2d, parts, W1a, b1a.reshape(1, 512), g1.reshape(1, 512),
    be1.reshape(1, 512), Wf, bf)


def _tc2(c1, agg, b2a, g2, be2, W2b, b2b, Wl1, bl1, Wl2, bl2):
  def body(c1_ref, agg_ref, b2a_ref, g2_ref, be2_ref, w2b_ref, b2b_ref,
           wl1_ref, bl1_ref, wl2_ref, bl2_ref, out_ref):
    u = c1_ref[...] + agg_ref[...] + b2a_ref[...]
    v = jnp.maximum(u * (g2_ref[...] * _BN_SCALE) + be2_ref[...], 0.0)
    h2 = (jnp.dot(v, w2b_ref[...], preferred_element_type=jnp.float32)
          + b2b_ref[...])
    q = jnp.maximum(
        jnp.dot(h2, wl1_ref[...], preferred_element_type=jnp.float32)
        + bl1_ref[...], 0.0)
    o = (jnp.dot(q, wl2_ref[...], preferred_element_type=jnp.float32)
         + bl2_ref[...])
    m = jnp.max(o, axis=1, keepdims=True)
    lse = m + jnp.log(jnp.sum(jnp.exp(o - m), axis=1, keepdims=True))
    out_ref[...] = o - lse

  full = lambda shape: pl.BlockSpec(shape, lambda i: (0, 0))
  return pl.pallas_call(
      body,
      grid=(NBLK,),
      in_specs=[
          pl.BlockSpec((BLK, 128), lambda i: (i, 0)),
          pl.BlockSpec((BLK, 128), lambda i: (i, 0)),
          full((1, 128)),
          full((1, 128)),
          full((1, 128)),
          full((128, 64)),
          full((1, 64)),
          full((64, 16)),
          full((1, 16)),
          full((16, 6)),
          full((1, 6)),
      ],
      out_specs=pl.BlockSpec((BLK, 6), lambda i: (i, 0)),
      out_shape=jax.ShapeDtypeStruct((NP, 6), jnp.float32),
  )(c1, agg, b2a.reshape(1, 128), g2.reshape(1, 128), be2.reshape(1, 128),
    W2b, b2b.reshape(1, 64), Wl1, bl1.reshape(1, 16), Wl2, bl2.reshape(1, 6))


def kernel(x, edge_attr, W1a, b1a, g1, be1, W1b, b1b, W2a, b2a, g2, be2,
           W2b, b2b, Wl1, bl1, Wl2, bl2, edge_index):
  del edge_attr
  src = edge_index[0]
  dst = edge_index[1]
  padlen = EPB - E
  srcp = jnp.concatenate([src, jnp.zeros((padlen,), jnp.int32)])
  dstp = jnp.concatenate([
      dst,
      (1 << 30) + (jnp.arange(padlen, dtype=jnp.int32) & 127),
  ])
  dst2d = dstp.reshape(EPB // CKS, CKS)
  xp = jnp.concatenate([x[:, 0], jnp.zeros((NP - N,), jnp.float32)])

  parts = _segsum_scalar(srcp, dstp, xp)
  parts2 = parts.reshape(2, AR * 128)[:, :NP]

  Wf, bf = _fuse_weights(W1b, W2a, b1b)
  c1 = _tc1(xp.reshape(NP, 1), parts2, W1a, b1a, g1, be1, Wf, bf)

  zeros_hbm = jnp.zeros((ZPT, 128), jnp.float32)
  agg = _segsum_vec(srcp, dst2d, zeros_hbm, c1)[:NP]

  out = _tc2(c1, agg, b2a, g2, be2, W2b, b2b, Wl1, bl1, Wl2, bl2)
  return out[:N]


# speedup vs baseline: 3.6930x; 3.6930x over previous
import functools

import jax
import jax.numpy as jnp
from jax import lax
from jax.experimental import pallas as pl
from jax.experimental.pallas import tpu as pltpu
from jax.experimental.pallas import tpu_sc as plsc

N = 50000
E = 800000
NP = 50176
BLK = 512
NBLK = NP // BLK

AR = 512
ATRASH = 65024

SUP = 2048
CKS = 64
NCK = SUP // CKS
NBUF = 2
GRP = NBUF * CKS
CCAP = SUP + GRP
NSUP = 26
EPT_B = NSUP * SUP
EPB = 16 * EPT_B
EPT_A = EPB // 32
CHA = 2048
NCH_A = EPT_A // CHA

NPASS = 2
RANGE = 12544
OUTR = 2 * NPASS * RANGE
TRASH = 256
SR = RANGE + TRASH
ZPT = SR // 16

_BN_SCALE = 1.0 / (1.0 + 1e-5) ** 0.5


def _segsum_scalar(srcp, dstp, xp):
  mesh = plsc.VectorSubcoreMesh(core_axis_name="c", subcore_axis_name="s")

  @functools.partial(
      pl.kernel,
      out_type=jax.ShapeDtypeStruct((2, AR, 128), jnp.float32),
      mesh=mesh,
      compiler_params=pltpu.CompilerParams(needs_layout_passes=False),
      scratch_types=[
          pltpu.VMEM((NP // 128, 128), jnp.float32),
          pltpu.VMEM((AR, 128), jnp.float32),
          pltpu.VMEM((CHA,), jnp.int32),
          pltpu.VMEM((CHA,), jnp.int32),
          pltpu.VMEM((4, 128), jnp.int32),
          pltpu.VMEM_SHARED((AR, 128), jnp.float32),
      ],
  )
  def body(src_hbm, dst_hbm, x_hbm, out_hbm, x_v, acc_v, sv, dv, ramp, acc_sh):
    cid = lax.axis_index("c")
    sid = lax.axis_index("s")
    wid = sid * 2 + cid
    pltpu.sync_copy(x_hbm, x_v)

    zeros16 = jnp.zeros((16,), jnp.float32)

    def zero_row(r, carry):
      for q in range(8):
        acc_v[r, pl.ds(q * 16, 16)] = zeros16
      return carry

    lax.fori_loop(0, AR, zero_row, 0)

    for g in range(4):
      for j in range(8):
        ramp[g, pl.ds(j * 16, 16)] = lax.iota(jnp.int32, 16) + (g * 128 + j * 16)

    ebase = wid * EPT_A

    def chunk(ch, carry):
      off = pl.multiple_of(ebase + ch * CHA, 8)
      pltpu.sync_copy(src_hbm.at[pl.ds(off, CHA)], sv)
      pltpu.sync_copy(dst_hbm.at[pl.ds(off, CHA)], dv)

      def inner(j, c2):
        s = sv[pl.ds(j * 16, 16)]
        d = dv[pl.ds(j * 16, 16)]
        d = jnp.where(d >= NP, ATRASH + jnp.bitwise_and(d, 127), d)
        vals = plsc.load_gather(
            x_v, [lax.shift_right_logical(s, 7), jnp.bitwise_and(s, 127)])
        plsc.addupdate_scatter(
            acc_v,
            [lax.shift_right_logical(d, 7), jnp.bitwise_and(d, 127)],
            vals,
        )
        return c2

      lax.fori_loop(0, CHA // 16, inner, 0)
      return carry

    lax.fori_loop(0, NCH_A, chunk, 0)

    @pl.when(sid == 0)
    def _():
      pltpu.sync_copy(acc_v, acc_sh)

    plsc.subcore_barrier()

    @pl.when(sid != 0)
    def _():
      for g in range(4):
        pltpu.sync_copy(
            acc_v.at[pl.ds(g * 128, 128)], acc_sh.at[ramp.at[g]], add=True
        )

    plsc.subcore_barrier()
    rows = AR // 16
    roff = pl.multiple_of(sid * rows, 8)
    pltpu.sync_copy(
        acc_sh.at[pl.ds(roff, rows)],
        out_hbm.at[cid, pl.ds(roff, rows)],
    )

  return body(srcp, dstp, xp.reshape(NP // 128, 128))


def _segsum_vec(srcp, dst2d, zeros_hbm, c1):
  mesh = plsc.VectorSubcoreMesh(core_axis_name="c", subcore_axis_name="s")

  @functools.partial(
      pl.kernel,
      out_type=jax.ShapeDtypeStruct((OUTR, 128), jnp.float32),
      mesh=mesh,
      compiler_params=pltpu.CompilerParams(needs_layout_passes=False),
      scratch_types=[
          pltpu.VMEM((SUP,), jnp.int32),
          pltpu.VMEM((NCK, CKS), jnp.int32),
          pltpu.VMEM((CCAP,), jnp.int32),
          pltpu.VMEM((CCAP,), jnp.int32),
          pltpu.VMEM((CKS, 128), jnp.float32),
          pltpu.VMEM((CKS, 128), jnp.float32),
          pltpu.VMEM((NBUF, CKS), jnp.int32),
          pltpu.VMEM_SHARED((SR, 128), jnp.float32),
          [pltpu.SemaphoreType.DMA] * NBUF,
          [pltpu.SemaphoreType.DMA] * NBUF,
      ],
  )
  def body(src_hbm, dst_hbm, z_hbm, c1_hbm, out_hbm, sv, dv, cs, cd,
           rows0, rows1, cd2, acc_sh, gsems, ssems):
    rows = [rows0, rows1]
    cid = lax.axis_index("c")
    sid = lax.axis_index("s")
    tb = sid * EPT_B

    def zcs(i, c):
      cs[pl.ds(i * 16, 16)] = jnp.zeros((16,), jnp.int32)
      return c
    lax.fori_loop(0, CCAP // 16, zcs, 0)

    for p in range(NPASS):
      base = (p * 2 + cid) * RANGE
      zoff = pl.multiple_of(sid * ZPT, 8)
      pltpu.sync_copy(z_hbm, acc_sh.at[pl.ds(zoff, ZPT)])
      plsc.subcore_barrier()

      def sup_body(sc, carry):
        off = pl.multiple_of(tb + sc * SUP, 8)
        pltpu.sync_copy(src_hbm.at[pl.ds(off, SUP)], sv)
        pltpu.sync_copy(dst_hbm.at[pl.ds(pl.multiple_of(off // CKS, 8), NCK)],
                        dv)

        def compact(g, cnt):
          r = g // (CKS // 16)
          q = g % (CKS // 16)
          d = dv[r, pl.ds(q * 16, 16)]
          s = sv[pl.ds(g * 16, 16)]
          l = d - base
          m = (l >= 0) & (l < RANGE)
          plsc.store_compressed(cs.at[pl.ds(cnt, 16)], s, mask=m)
          plsc.store_compressed(cd.at[pl.ds(cnt, 16)], l, mask=m)
          return cnt + jnp.max(plsc.all_reduce_population_count(m))

        cnt = lax.fori_loop(0, SUP // 16, compact, 0)

        tr16 = RANGE + lax.iota(jnp.int32, 16)
        for t in range(GRP // 16):
          cs[pl.ds(cnt + t * 16, 16)] = jnp.zeros((16,), jnp.int32)
          cd[pl.ds(cnt + t * 16, 16)] = tr16 + 16 * t

        ngroups = (cnt + GRP - 1) // GRP

        def gidx(grp, b):
          return pl.multiple_of(grp * GRP + b * CKS, 8)

        def grp_body(k4, c2):
          gs = [
              pltpu.async_copy(
                  c1_hbm.at[cs.at[pl.ds(gidx(k4, b), CKS)]], rows[b],
                  gsems[b])
              for b in range(NBUF)
          ]
          ss = []
          for b in range(NBUF):
            gs[b].wait()
            for j in range(CKS // 16):
              cd2[b, pl.ds(j * 16, 16)] = cd[pl.ds(gidx(k4, b) + j * 16, 16)]
            ss.append(pltpu.async_copy(
                rows[b], acc_sh.at[cd2.at[b]], ssems[b], add=True))
          for s in ss:
            s.wait()
          return c2

        lax.fori_loop(0, ngroups, grp_body, 0)
        return carry

      lax.fori_loop(0, NSUP, sup_body, 0)
      plsc.subcore_barrier()
      doff = pl.multiple_of(sid * (RANGE // 16), 8)
      pltpu.sync_copy(
          acc_sh.at[pl.ds(doff, RANGE // 16)],
          out_hbm.at[pl.ds(pl.multiple_of(base + doff, 8), RANGE // 16)],
      )
      plsc.subcore_barrier()

  return body(srcp, dst2d, zeros_hbm, c1)


def _fuse_weights(W1b, W2a, b1b):
  def body(w1b_ref, w2a_ref, b1b_ref, wf_ref, bf_ref):
    wf_ref[...] = jnp.dot(w1b_ref[...], w2a_ref[...],
                          preferred_element_type=jnp.float32)
    bf_ref[...] = jnp.dot(b1b_ref[...], w2a_ref[...],
                          preferred_element_type=jnp.float32)

  return pl.pallas_call(
      body,
      out_shape=[
          jax.ShapeDtypeStruct((512, 128), jnp.float32),
          jax.ShapeDtypeStruct((1, 128), jnp.float32),
      ],
  )(W1b, W2a, b1b.reshape(1, 256))


def _tc1(xp2d, parts, W1a, b1a, g1, be1, Wf, bf):
  def body(x_ref, pp_ref, w1a_ref, b1a_ref, g1_ref, be1_ref, wf_ref, bf_ref,
           out_ref):
    pp = pp_ref[...]
    z = x_ref[...] + pp[0][:, None] + pp[1][:, None]
    t = z * w1a_ref[...] + b1a_ref[...]
    r = jnp.maximum(t * (g1_ref[...] * _BN_SCALE) + be1_ref[...], 0.0)
    out_ref[...] = (
        jnp.dot(r, wf_ref[...], preferred_element_type=jnp.float32)
        + bf_ref[...]
    )

  full = lambda shape: pl.BlockSpec(shape, lambda i: (0, 0))
  return pl.pallas_call(
      body,
      grid=(NBLK,),
      in_specs=[
          pl.BlockSpec((BLK, 1), lambda i: (i, 0)),
          pl.BlockSpec((2, BLK), lambda i: (0, i)),
          full((1, 512)),
          full((1, 512)),
          full((1, 512)),
          full((1, 512)),
          full((512, 128)),
          full((1, 128)),
      ],
      out_specs=pl.BlockSpec((BLK, 128), lambda i: (i, 0)),
      out_shape=jax.ShapeDtypeStruct((NP, 128), jnp.float32),
  )(xp2d, parts, W1a, b1a.reshape(1, 512), g1.reshape(1, 512),
    be1.reshape(1, 512), Wf, bf)


def _tc2(c1, agg, b2a, g2, be2, W2b, b2b, Wl1, bl1, Wl2, bl2):
  def body(c1_ref, agg_ref, b2a_ref, g2_ref, be2_ref, w2b_ref, b2b_ref,
           wl1_ref, bl1_ref, wl2_ref, bl2_ref, out_ref):
    u = c1_ref[...] + agg_ref[...] + b2a_ref[...]
    v = jnp.maximum(u * (g2_ref[...] * _BN_SCALE) + be2_ref[...], 0.0)
    h2 = (jnp.dot(v, w2b_ref[...], preferred_element_type=jnp.float32)
          + b2b_ref[...])
    q = jnp.maximum(
        jnp.dot(h2, wl1_ref[...], preferred_element_type=jnp.float32)
        + bl1_ref[...], 0.0)
    o = (jnp.dot(q, wl2_ref[...], preferred_element_type=jnp.float32)
         + bl2_ref[...])
    m = jnp.max(o, axis=1, keepdims=True)
    lse = m + jnp.log(jnp.sum(jnp.exp(o - m), axis=1, keepdims=True))
    out_ref[...] = o - lse

  full = lambda shape: pl.BlockSpec(shape, lambda i: (0, 0))
  return pl.pallas_call(
      body,
      grid=(NBLK,),
      in_specs=[
          pl.BlockSpec((BLK, 128), lambda i: (i, 0)),
          pl.BlockSpec((BLK, 128), lambda i: (i, 0)),
          full((1, 128)),
          full((1, 128)),
          full((1, 128)),
          full((128, 64)),
          full((1, 64)),
          full((64, 16)),
          full((1, 16)),
          full((16, 6)),
          full((1, 6)),
      ],
      out_specs=pl.BlockSpec((BLK, 6), lambda i: (i, 0)),
      out_shape=jax.ShapeDtypeStruct((NP, 6), jnp.float32),
  )(c1, agg, b2a.reshape(1, 128), g2.reshape(1, 128), be2.reshape(1, 128),
    W2b, b2b.reshape(1, 64), Wl1, bl1.reshape(1, 16), Wl2, bl2.reshape(1, 6))


def kernel(x, edge_attr, W1a, b1a, g1, be1, W1b, b1b, W2a, b2a, g2, be2,
           W2b, b2b, Wl1, bl1, Wl2, bl2, edge_index):
  del edge_attr
  src = edge_index[0]
  dst = edge_index[1]
  padlen = EPB - E
  srcp = jnp.concatenate([src, jnp.zeros((padlen,), jnp.int32)])
  dstp = jnp.concatenate([
      dst,
      (1 << 30) + (jnp.arange(padlen, dtype=jnp.int32) & 127),
  ])
  dst2d = dstp.reshape(EPB // CKS, CKS)
  xp = jnp.concatenate([x[:, 0], jnp.zeros((NP - N,), jnp.float32)])

  parts = _segsum_scalar(srcp, dstp, xp)
  parts2 = parts.reshape(2, AR * 128)[:, :NP]

  Wf, bf = _fuse_weights(W1b, W2a, b1b)
  c1 = _tc1(xp.reshape(NP, 1), parts2, W1a, b1a, g1, be1, Wf, bf)

  zeros_hbm = jnp.zeros((ZPT, 128), jnp.float32)
  agg = _segsum_vec(srcp, dst2d, zeros_hbm, c1)[:NP]

  out = _tc2(c1, agg, b2a, g2, be2, W2b, b2b, Wl1, bl1, Wl2, bl2)
  return out[:N]


# speedup vs baseline: 6.4333x; 1.7420x over previous
import functools

import jax
import jax.numpy as jnp
from jax import lax
from jax.experimental import pallas as pl
from jax.experimental.pallas import tpu as pltpu
from jax.experimental.pallas import tpu_sc as plsc

N = 50000
E = 800000
NP = 50176
BLK = 512
NBLK = NP // BLK

AR = 512
ATRASH = 65024

SUP = 2048
CKS = 32
NCK = SUP // CKS
NBUF = 2
GRP = NBUF * CKS
CCAP = SUP + GRP
NSUP = 26
EPT_B = NSUP * SUP
EPB = 16 * EPT_B
EPT_A = EPB // 32
CHA = 2048
NCH_A = EPT_A // CHA

NPASS = 2
RANGE = 12544
OUTR = 2 * NPASS * RANGE
TRASH = 256
SR = RANGE + TRASH
ZPT = SR // 16

_BN_SCALE = 1.0 / (1.0 + 1e-5) ** 0.5


def _segsum_scalar(srcp, dstp, xp):
  mesh = plsc.VectorSubcoreMesh(core_axis_name="c", subcore_axis_name="s")

  @functools.partial(
      pl.kernel,
      out_type=jax.ShapeDtypeStruct((2, AR, 128), jnp.float32),
      mesh=mesh,
      compiler_params=pltpu.CompilerParams(needs_layout_passes=False),
      scratch_types=[
          pltpu.VMEM((NP // 128, 128), jnp.float32),
          pltpu.VMEM((AR, 128), jnp.float32),
          pltpu.VMEM((CHA,), jnp.int32),
          pltpu.VMEM((CHA,), jnp.int32),
          pltpu.VMEM((4, 128), jnp.int32),
          pltpu.VMEM_SHARED((AR, 128), jnp.float32),
      ],
  )
  def body(src_hbm, dst_hbm, x_hbm, out_hbm, x_v, acc_v, sv, dv, ramp, acc_sh):
    cid = lax.axis_index("c")
    sid = lax.axis_index("s")
    wid = sid * 2 + cid
    pltpu.sync_copy(x_hbm, x_v)

    zeros16 = jnp.zeros((16,), jnp.float32)

    def zero_row(r, carry):
      for q in range(8):
        acc_v[r, pl.ds(q * 16, 16)] = zeros16
      return carry

    lax.fori_loop(0, AR, zero_row, 0)

    for g in range(4):
      for j in range(8):
        ramp[g, pl.ds(j * 16, 16)] = lax.iota(jnp.int32, 16) + (g * 128 + j * 16)

    ebase = wid * EPT_A

    def chunk(ch, carry):
      off = pl.multiple_of(ebase + ch * CHA, 8)
      pltpu.sync_copy(src_hbm.at[pl.ds(off, CHA)], sv)
      pltpu.sync_copy(dst_hbm.at[pl.ds(off, CHA)], dv)

      def inner(j, c2):
        s = sv[pl.ds(j * 16, 16)]
        d = dv[pl.ds(j * 16, 16)]
        d = jnp.where(d >= NP, ATRASH + jnp.bitwise_and(d, 127), d)
        vals = plsc.load_gather(
            x_v, [lax.shift_right_logical(s, 7), jnp.bitwise_and(s, 127)])
        plsc.addupdate_scatter(
            acc_v,
            [lax.shift_right_logical(d, 7), jnp.bitwise_and(d, 127)],
            vals,
        )
        return c2

      lax.fori_loop(0, CHA // 16, inner, 0)
      return carry

    lax.fori_loop(0, NCH_A, chunk, 0)

    @pl.when(sid == 0)
    def _():
      pltpu.sync_copy(acc_v, acc_sh)

    plsc.subcore_barrier()

    @pl.when(sid != 0)
    def _():
      for g in range(4):
        pltpu.sync_copy(
            acc_v.at[pl.ds(g * 128, 128)], acc_sh.at[ramp.at[g]], add=True
        )

    plsc.subcore_barrier()
    rows = AR // 16
    roff = pl.multiple_of(sid * rows, 8)
    pltpu.sync_copy(
        acc_sh.at[pl.ds(roff, rows)],
        out_hbm.at[cid, pl.ds(roff, rows)],
    )

  return body(srcp, dstp, xp.reshape(NP // 128, 128))


def _segsum_vec(srcp, dst2d, zeros_hbm, c1):
  mesh = plsc.VectorSubcoreMesh(core_axis_name="c", subcore_axis_name="s")

  @functools.partial(
      pl.kernel,
      out_type=jax.ShapeDtypeStruct((OUTR, 128), jnp.float32),
      mesh=mesh,
      compiler_params=pltpu.CompilerParams(needs_layout_passes=False),
      scratch_types=[
          pltpu.VMEM((SUP,), jnp.int32),
          pltpu.VMEM((NCK, CKS), jnp.int32),
          pltpu.VMEM((CCAP,), jnp.int32),
          pltpu.VMEM((CCAP,), jnp.int32),
          pltpu.VMEM((CKS, 128), jnp.float32),
          pltpu.VMEM((CKS, 128), jnp.float32),
          pltpu.VMEM((NBUF, CKS), jnp.int32),
          pltpu.VMEM_SHARED((SR, 128), jnp.float32),
          [pltpu.SemaphoreType.DMA] * NBUF,
          [pltpu.SemaphoreType.DMA] * NBUF,
      ],
  )
  def body(src_hbm, dst_hbm, z_hbm, c1_hbm, out_hbm, sv, dv, cs, cd,
           rows0, rows1, cd2, acc_sh, gsems, ssems):
    rows = [rows0, rows1]
    cid = lax.axis_index("c")
    sid = lax.axis_index("s")
    tb = sid * EPT_B

    def zcs(i, c):
      cs[pl.ds(i * 16, 16)] = jnp.zeros((16,), jnp.int32)
      return c
    lax.fori_loop(0, CCAP // 16, zcs, 0)

    for p in range(NPASS):
      base = (p * 2 + cid) * RANGE
      zoff = pl.multiple_of(sid * ZPT, 8)
      pltpu.sync_copy(z_hbm, acc_sh.at[pl.ds(zoff, ZPT)])
      plsc.subcore_barrier()

      def sup_body(sc, carry):
        off = pl.multiple_of(tb + sc * SUP, 8)
        pltpu.sync_copy(src_hbm.at[pl.ds(off, SUP)], sv)
        pltpu.sync_copy(dst_hbm.at[pl.ds(pl.multiple_of(off // CKS, 8), NCK)],
                        dv)

        def compact(g, cnt):
          r = g // (CKS // 16)
          q = g % (CKS // 16)
          d = dv[r, pl.ds(q * 16, 16)]
          s = sv[pl.ds(g * 16, 16)]
          l = d - base
          m = (l >= 0) & (l < RANGE)
          plsc.store_compressed(cs.at[pl.ds(cnt, 16)], s, mask=m)
          plsc.store_compressed(cd.at[pl.ds(cnt, 16)], l, mask=m)
          return cnt + jnp.max(plsc.all_reduce_population_count(m))

        cnt = lax.fori_loop(0, SUP // 16, compact, 0)

        tr16 = RANGE + lax.iota(jnp.int32, 16)
        for t in range(GRP // 16):
          cs[pl.ds(cnt + t * 16, 16)] = jnp.zeros((16,), jnp.int32)
          cd[pl.ds(cnt + t * 16, 16)] = tr16 + 16 * t

        ngroups = (cnt + GRP - 1) // GRP

        def gidx(grp, b):
          return pl.multiple_of(grp * GRP + b * CKS, 8)

        def grp_body(k4, c2):
          gs = [
              pltpu.async_copy(
                  c1_hbm.at[cs.at[pl.ds(gidx(k4, b), CKS)]], rows[b],
                  gsems[b])
              for b in range(NBUF)
          ]
          ss = []
          for b in range(NBUF):
            gs[b].wait()
            for j in range(CKS // 16):
              cd2[b, pl.ds(j * 16, 16)] = cd[pl.ds(gidx(k4, b) + j * 16, 16)]
            ss.append(pltpu.async_copy(
                rows[b], acc_sh.at[cd2.at[b]], ssems[b], add=True))
          for s in ss:
            s.wait()
          return c2

        lax.fori_loop(0, ngroups, grp_body, 0)
        return carry

      lax.fori_loop(0, NSUP, sup_body, 0)
      plsc.subcore_barrier()
      doff = pl.multiple_of(sid * (RANGE // 16), 8)
      pltpu.sync_copy(
          acc_sh.at[pl.ds(doff, RANGE // 16)],
          out_hbm.at[pl.ds(pl.multiple_of(base + doff, 8), RANGE // 16)],
      )
      plsc.subcore_barrier()

  return body(srcp, dst2d, zeros_hbm, c1)


def _fuse_weights(W1b, W2a, b1b):
  def body(w1b_ref, w2a_ref, b1b_ref, wf_ref, bf_ref):
    wf_ref[...] = jnp.dot(w1b_ref[...], w2a_ref[...],
                          preferred_element_type=jnp.float32)
    bf_ref[...] = jnp.dot(b1b_ref[...], w2a_ref[...],
                          preferred_element_type=jnp.float32)

  return pl.pallas_call(
      body,
      out_shape=[
          jax.ShapeDtypeStruct((512, 128), jnp.float32),
          jax.ShapeDtypeStruct((1, 128), jnp.float32),
      ],
  )(W1b, W2a, b1b.reshape(1, 256))


def _tc1(xp2d, parts, W1a, b1a, g1, be1, Wf, bf):
  def body(x_ref, pp_ref, w1a_ref, b1a_ref, g1_ref, be1_ref, wf_ref, bf_ref,
           out_ref):
    pp = pp_ref[...]
    z = x_ref[...] + pp[0][:, None] + pp[1][:, None]
    t = z * w1a_ref[...] + b1a_ref[...]
    r = jnp.maximum(t * (g1_ref[...] * _BN_SCALE) + be1_ref[...], 0.0)
    out_ref[...] = (
        jnp.dot(r, wf_ref[...], preferred_element_type=jnp.float32)
        + bf_ref[...]
    )

  full = lambda shape: pl.BlockSpec(shape, lambda i: (0, 0))
  return pl.pallas_call(
      body,
      grid=(NBLK,),
      in_specs=[
          pl.BlockSpec((BLK, 1), lambda i: (i, 0)),
          pl.BlockSpec((2, BLK), lambda i: (0, i)),
          full((1, 512)),
          full((1, 512)),
          full((1, 512)),
          full((1, 512)),
          full((512, 128)),
          full((1, 128)),
      ],
      out_specs=pl.BlockSpec((BLK, 128), lambda i: (i, 0)),
      out_shape=jax.ShapeDtypeStruct((NP, 128), jnp.float32),
  )(xp2d, parts, W1a, b1a.reshape(1, 512), g1.reshape(1, 512),
    be1.reshape(1, 512), Wf, bf)


def _tc2(c1, agg, b2a, g2, be2, W2b, b2b, Wl1, bl1, Wl2, bl2):
  def body(c1_ref, agg_ref, b2a_ref, g2_ref, be2_ref, w2b_ref, b2b_ref,
           wl1_ref, bl1_ref, wl2_ref, bl2_ref, out_ref):
    u = c1_ref[...] + agg_ref[...] + b2a_ref[...]
    v = jnp.maximum(u * (g2_ref[...] * _BN_SCALE) + be2_ref[...], 0.0)
    h2 = (jnp.dot(v, w2b_ref[...], preferred_element_type=jnp.float32)
          + b2b_ref[...])
    q = jnp.maximum(
        jnp.dot(h2, wl1_ref[...], preferred_element_type=jnp.float32)
        + bl1_ref[...], 0.0)
    o = (jnp.dot(q, wl2_ref[...], preferred_element_type=jnp.float32)
         + bl2_ref[...])
    m = jnp.max(o, axis=1, keepdims=True)
    lse = m + jnp.log(jnp.sum(jnp.exp(o - m), axis=1, keepdims=True))
    out_ref[...] = o - lse

  full = lambda shape: pl.BlockSpec(shape, lambda i: (0, 0))
  return pl.pallas_call(
      body,
      grid=(NBLK,),
      in_specs=[
          pl.BlockSpec((BLK, 128), lambda i: (i, 0)),
          pl.BlockSpec((BLK, 128), lambda i: (i, 0)),
          full((1, 128)),
          full((1, 128)),
          full((1, 128)),
          full((128, 64)),
          full((1, 64)),
          full((64, 16)),
          full((1, 16)),
          full((16, 6)),
          full((1, 6)),
      ],
      out_specs=pl.BlockSpec((BLK, 6), lambda i: (i, 0)),
      out_shape=jax.ShapeDtypeStruct((NP, 6), jnp.float32),
  )(c1, agg, b2a.reshape(1, 128), g2.reshape(1, 128), be2.reshape(1, 128),
    W2b, b2b.reshape(1, 64), Wl1, bl1.reshape(1, 16), Wl2, bl2.reshape(1, 6))


def kernel(x, edge_attr, W1a, b1a, g1, be1, W1b, b1b, W2a, b2a, g2, be2,
           W2b, b2b, Wl1, bl1, Wl2, bl2, edge_index):
  del edge_attr
  src = edge_index[0]
  dst = edge_index[1]
  padlen = EPB - E
  srcp = jnp.concatenate([src, jnp.zeros((padlen,), jnp.int32)])
  dstp = jnp.concatenate([
      dst,
      (1 << 30) + (jnp.arange(padlen, dtype=jnp.int32) & 127),
  ])
  dst2d = dstp.reshape(EPB // CKS, CKS)
  xp = jnp.concatenate([x[:, 0], jnp.zeros((NP - N,), jnp.float32)])

  parts = _segsum_scalar(srcp, dstp, xp)
  parts2 = parts.reshape(2, AR * 128)[:, :NP]

  Wf, bf = _fuse_weights(W1b, W2a, b1b)
  c1 = _tc1(xp.reshape(NP, 1), parts2, W1a, b1a, g1, be1, Wf, bf)

  zeros_hbm = jnp.zeros((ZPT, 128), jnp.float32)
  agg = _segsum_vec(srcp, dst2d, zeros_hbm, c1)[:NP]

  out = _tc2(c1, agg, b2a, g2, be2, W2b, b2b, Wl1, bl1, Wl2, bl2)
  return out[:N]


# speedup vs baseline: 9.6838x; 1.5053x over previous
import functools

import jax
import jax.numpy as jnp
from jax import lax
from jax.experimental import pallas as pl
from jax.experimental.pallas import tpu as pltpu
from jax.experimental.pallas import tpu_sc as plsc

N = 50000
E = 800000
NP = 50176
BLK = 512
NBLK = NP // BLK

AR = 512
ATRASH = 65024

SUP = 2048
CKS = 16
NCK = SUP // CKS
NBUF = 2
GRP = NBUF * CKS
CCAP = SUP + GRP
NSUP = 26
EPT_B = NSUP * SUP
EPB = 16 * EPT_B
EPT_A = EPB // 32
CHA = 2048
NCH_A = EPT_A // CHA

NPASS = 2
RANGE = 12544
OUTR = 2 * NPASS * RANGE
TRASH = 256
SR = RANGE + TRASH
ZPT = SR // 16

_BN_SCALE = 1.0 / (1.0 + 1e-5) ** 0.5


def _segsum_scalar(srcp, dstp, xp):
  mesh = plsc.VectorSubcoreMesh(core_axis_name="c", subcore_axis_name="s")

  @functools.partial(
      pl.kernel,
      out_type=jax.ShapeDtypeStruct((2, AR, 128), jnp.float32),
      mesh=mesh,
      compiler_params=pltpu.CompilerParams(needs_layout_passes=False),
      scratch_types=[
          pltpu.VMEM((NP // 128, 128), jnp.float32),
          pltpu.VMEM((AR, 128), jnp.float32),
          pltpu.VMEM((CHA,), jnp.int32),
          pltpu.VMEM((CHA,), jnp.int32),
          pltpu.VMEM((4, 128), jnp.int32),
          pltpu.VMEM_SHARED((AR, 128), jnp.float32),
      ],
  )
  def body(src_hbm, dst_hbm, x_hbm, out_hbm, x_v, acc_v, sv, dv, ramp, acc_sh):
    cid = lax.axis_index("c")
    sid = lax.axis_index("s")
    wid = sid * 2 + cid
    pltpu.sync_copy(x_hbm, x_v)

    zeros16 = jnp.zeros((16,), jnp.float32)

    def zero_row(r, carry):
      for q in range(8):
        acc_v[r, pl.ds(q * 16, 16)] = zeros16
      return carry

    lax.fori_loop(0, AR, zero_row, 0)

    for g in range(4):
      for j in range(8):
        ramp[g, pl.ds(j * 16, 16)] = lax.iota(jnp.int32, 16) + (g * 128 + j * 16)

    ebase = wid * EPT_A

    def chunk(ch, carry):
      off = pl.multiple_of(ebase + ch * CHA, 8)
      pltpu.sync_copy(src_hbm.at[pl.ds(off, CHA)], sv)
      pltpu.sync_copy(dst_hbm.at[pl.ds(off, CHA)], dv)

      def inner(j, c2):
        s = sv[pl.ds(j * 16, 16)]
        d = dv[pl.ds(j * 16, 16)]
        d = jnp.where(d >= NP, ATRASH + jnp.bitwise_and(d, 127), d)
        vals = plsc.load_gather(
            x_v, [lax.shift_right_logical(s, 7), jnp.bitwise_and(s, 127)])
        plsc.addupdate_scatter(
            acc_v,
            [lax.shift_right_logical(d, 7), jnp.bitwise_and(d, 127)],
            vals,
        )
        return c2

      lax.fori_loop(0, CHA // 16, inner, 0)
      return carry

    lax.fori_loop(0, NCH_A, chunk, 0)

    @pl.when(sid == 0)
    def _():
      pltpu.sync_copy(acc_v, acc_sh)

    plsc.subcore_barrier()

    @pl.when(sid != 0)
    def _():
      for g in range(4):
        pltpu.sync_copy(
            acc_v.at[pl.ds(g * 128, 128)], acc_sh.at[ramp.at[g]], add=True
        )

    plsc.subcore_barrier()
    rows = AR // 16
    roff = pl.multiple_of(sid * rows, 8)
    pltpu.sync_copy(
        acc_sh.at[pl.ds(roff, rows)],
        out_hbm.at[cid, pl.ds(roff, rows)],
    )

  return body(srcp, dstp, xp.reshape(NP // 128, 128))


def _segsum_vec(srcp, dst2d, zeros_hbm, c1):
  mesh = plsc.VectorSubcoreMesh(core_axis_name="c", subcore_axis_name="s")

  @functools.partial(
      pl.kernel,
      out_type=jax.ShapeDtypeStruct((OUTR, 128), jnp.float32),
      mesh=mesh,
      compiler_params=pltpu.CompilerParams(needs_layout_passes=False),
      scratch_types=[
          pltpu.VMEM((SUP,), jnp.int32),
          pltpu.VMEM((NCK, CKS), jnp.int32),
          pltpu.VMEM((CCAP,), jnp.int32),
          pltpu.VMEM((CCAP,), jnp.int32),
          pltpu.VMEM((CKS, 128), jnp.float32),
          pltpu.VMEM((CKS, 128), jnp.float32),
          pltpu.VMEM((NBUF, CKS), jnp.int32),
          pltpu.VMEM_SHARED((SR, 128), jnp.float32),
          [pltpu.SemaphoreType.DMA] * NBUF,
          [pltpu.SemaphoreType.DMA] * NBUF,
      ],
  )
  def body(src_hbm, dst_hbm, z_hbm, c1_hbm, out_hbm, sv, dv, cs, cd,
           rows0, rows1, cd2, acc_sh, gsems, ssems):
    rows = [rows0, rows1]
    cid = lax.axis_index("c")
    sid = lax.axis_index("s")
    tb = sid * EPT_B

    def zcs(i, c):
      cs[pl.ds(i * 16, 16)] = jnp.zeros((16,), jnp.int32)
      return c
    lax.fori_loop(0, CCAP // 16, zcs, 0)

    for p in range(NPASS):
      base = (p * 2 + cid) * RANGE
      zoff = pl.multiple_of(sid * ZPT, 8)
      pltpu.sync_copy(z_hbm, acc_sh.at[pl.ds(zoff, ZPT)])
      plsc.subcore_barrier()

      def sup_body(sc, carry):
        off = pl.multiple_of(tb + sc * SUP, 8)
        pltpu.sync_copy(src_hbm.at[pl.ds(off, SUP)], sv)
        pltpu.sync_copy(dst_hbm.at[pl.ds(pl.multiple_of(off // CKS, 8), NCK)],
                        dv)

        def compact(g, cnt):
          r = g // (CKS // 16)
          q = g % (CKS // 16)
          d = dv[r, pl.ds(q * 16, 16)]
          s = sv[pl.ds(g * 16, 16)]
          l = d - base
          m = (l >= 0) & (l < RANGE)
          plsc.store_compressed(cs.at[pl.ds(cnt, 16)], s, mask=m)
          plsc.store_compressed(cd.at[pl.ds(cnt, 16)], l, mask=m)
          return cnt + jnp.max(plsc.all_reduce_population_count(m))

        cnt = lax.fori_loop(0, SUP // 16, compact, 0)

        tr16 = RANGE + lax.iota(jnp.int32, 16)
        for t in range(GRP // 16):
          cs[pl.ds(cnt + t * 16, 16)] = jnp.zeros((16,), jnp.int32)
          cd[pl.ds(cnt + t * 16, 16)] = tr16 + 16 * t

        ngroups = (cnt + GRP - 1) // GRP

        def gidx(grp, b):
          return pl.multiple_of(grp * GRP + b * CKS, 8)

        def grp_body(k4, c2):
          gs = [
              pltpu.async_copy(
                  c1_hbm.at[cs.at[pl.ds(gidx(k4, b), CKS)]], rows[b],
                  gsems[b])
              for b in range(NBUF)
          ]
          ss = []
          for b in range(NBUF):
            gs[b].wait()
            for j in range(CKS // 16):
              cd2[b, pl.ds(j * 16, 16)] = cd[pl.ds(gidx(k4, b) + j * 16, 16)]
            ss.append(pltpu.async_copy(
                rows[b], acc_sh.at[cd2.at[b]], ssems[b], add=True))
          for s in ss:
            s.wait()
          return c2

        lax.fori_loop(0, ngroups, grp_body, 0)
        return carry

      lax.fori_loop(0, NSUP, sup_body, 0)
      plsc.subcore_barrier()
      doff = pl.multiple_of(sid * (RANGE // 16), 8)
      pltpu.sync_copy(
          acc_sh.at[pl.ds(doff, RANGE // 16)],
          out_hbm.at[pl.ds(pl.multiple_of(base + doff, 8), RANGE // 16)],
      )
      plsc.subcore_barrier()

  return body(srcp, dst2d, zeros_hbm, c1)


def _fuse_weights(W1b, W2a, b1b):
  def body(w1b_ref, w2a_ref, b1b_ref, wf_ref, bf_ref):
    wf_ref[...] = jnp.dot(w1b_ref[...], w2a_ref[...],
                          preferred_element_type=jnp.float32)
    bf_ref[...] = jnp.dot(b1b_ref[...], w2a_ref[...],
                          preferred_element_type=jnp.float32)

  return pl.pallas_call(
      body,
      out_shape=[
          jax.ShapeDtypeStruct((512, 128), jnp.float32),
          jax.ShapeDtypeStruct((1, 128), jnp.float32),
      ],
  )(W1b, W2a, b1b.reshape(1, 256))


def _tc1(xp2d, parts, W1a, b1a, g1, be1, Wf, bf):
  def body(x_ref, pp_ref, w1a_ref, b1a_ref, g1_ref, be1_ref, wf_ref, bf_ref,
           out_ref):
    pp = pp_ref[...]
    z = x_ref[...] + pp[0][:, None] + pp[1][:, None]
    t = z * w1a_ref[...] + b1a_ref[...]
    r = jnp.maximum(t * (g1_ref[...] * _BN_SCALE) + be1_ref[...], 0.0)
    out_ref[...] = (
        jnp.dot(r, wf_ref[...], preferred_element_type=jnp.float32)
        + bf_ref[...]
    )

  full = lambda shape: pl.BlockSpec(shape, lambda i: (0, 0))
  return pl.pallas_call(
      body,
      grid=(NBLK,),
      in_specs=[
          pl.BlockSpec((BLK, 1), lambda i: (i, 0)),
          pl.BlockSpec((2, BLK), lambda i: (0, i)),
          full((1, 512)),
          full((1, 512)),
          full((1, 512)),
          full((1, 512)),
          full((512, 128)),
          full((1, 128)),
      ],
      out_specs=pl.BlockSpec((BLK, 128), lambda i: (i, 0)),
      out_shape=jax.ShapeDtypeStruct((NP, 128), jnp.float32),
  )(xp2d, parts, W1a, b1a.reshape(1, 512), g1.reshape(1, 512),
    be1.reshape(1, 512), Wf, bf)


def _tc2(c1, agg, b2a, g2, be2, W2b, b2b, Wl1, bl1, Wl2, bl2):
  def body(c1_ref, agg_ref, b2a_ref, g2_ref, be2_ref, w2b_ref, b2b_ref,
           wl1_ref, bl1_ref, wl2_ref, bl2_ref, out_ref):
    u = c1_ref[...] + agg_ref[...] + b2a_ref[...]
    v = jnp.maximum(u * (g2_ref[...] * _BN_SCALE) + be2_ref[...], 0.0)
    h2 = (jnp.dot(v, w2b_ref[...], preferred_element_type=jnp.float32)
          + b2b_ref[...])
    q = jnp.maximum(
        jnp.dot(h2, wl1_ref[...], preferred_element_type=jnp.float32)
        + bl1_ref[...], 0.0)
    o = (jnp.dot(q, wl2_ref[...], preferred_element_type=jnp.float32)
         + bl2_ref[...])
    m = jnp.max(o, axis=1, keepdims=True)
    lse = m + jnp.log(jnp.sum(jnp.exp(o - m), axis=1, keepdims=True))
    out_ref[...] = o - lse

  full = lambda shape: pl.BlockSpec(shape, lambda i: (0, 0))
  return pl.pallas_call(
      body,
      grid=(NBLK,),
      in_specs=[
          pl.BlockSpec((BLK, 128), lambda i: (i, 0)),
          pl.BlockSpec((BLK, 128), lambda i: (i, 0)),
          full((1, 128)),
          full((1, 128)),
          full((1, 128)),
          full((128, 64)),
          full((1, 64)),
          full((64, 16)),
          full((1, 16)),
          full((16, 6)),
          full((1, 6)),
      ],
      out_specs=pl.BlockSpec((BLK, 6), lambda i: (i, 0)),
      out_shape=jax.ShapeDtypeStruct((NP, 6), jnp.float32),
  )(c1, agg, b2a.reshape(1, 128), g2.reshape(1, 128), be2.reshape(1, 128),
    W2b, b2b.reshape(1, 64), Wl1, bl1.reshape(1, 16), Wl2, bl2.reshape(1, 6))


def kernel(x, edge_attr, W1a, b1a, g1, be1, W1b, b1b, W2a, b2a, g2, be2,
           W2b, b2b, Wl1, bl1, Wl2, bl2, edge_index):
  del edge_attr
  src = edge_index[0]
  dst = edge_index[1]
  padlen = EPB - E
  srcp = jnp.concatenate([src, jnp.zeros((padlen,), jnp.int32)])
  dstp = jnp.concatenate([
      dst,
      (1 << 30) + (jnp.arange(padlen, dtype=jnp.int32) & 127),
  ])
  dst2d = dstp.reshape(EPB // CKS, CKS)
  xp = jnp.concatenate([x[:, 0], jnp.zeros((NP - N,), jnp.float32)])

  parts = _segsum_scalar(srcp, dstp, xp)
  parts2 = parts.reshape(2, AR * 128)[:, :NP]

  Wf, bf = _fuse_weights(W1b, W2a, b1b)
  c1 = _tc1(xp.reshape(NP, 1), parts2, W1a, b1a, g1, be1, Wf, bf)

  zeros_hbm = jnp.zeros((ZPT, 128), jnp.float32)
  agg = _segsum_vec(srcp, dst2d, zeros_hbm, c1)[:NP]

  out = _tc2(c1, agg, b2a, g2, be2, W2b, b2b, Wl1, bl1, Wl2, bl2)
  return out[:N]
